# baseline (speedup 1.0000x reference)
import functools

import jax
import jax.numpy as jnp
from jax import lax
from jax.experimental import pallas as pl
from jax.experimental.pallas import tpu as pltpu
from jax.experimental.pallas import tpu_sc as plsc

E = 8
TOPK = 2
C = 1024
INNER = 1024

BLK = 256
BT_R = 1024

NC, NS = 2, 16
NW = NC * NS
GCH = 96
CCH = 16


def _router_body(x_ref, wr_ref, br_ref, e0_ref, e1_ref, w0_ref, w1_ref):
    xb = x_ref[...]
    logits = lax.dot_general(
        xb, wr_ref[...], (((1,), (1,)), ((), ())),
        preferred_element_type=jnp.float32) + br_ref[...]
    p = jax.nn.softmax(logits, axis=-1)
    iota_e = lax.broadcasted_iota(jnp.int32, p.shape, 1)
    c1 = jnp.argmax(p, axis=-1)
    p1 = jnp.max(p, axis=-1)
    p_m = jnp.where(iota_e == c1[:, None], -jnp.inf, p)
    c2 = jnp.argmax(p_m, axis=-1)
    p2 = jnp.max(p_m, axis=-1)
    e0_ref[...] = c1[:, None].astype(jnp.int32)
    e1_ref[...] = c2[:, None].astype(jnp.int32)
    w0_ref[...] = p1[:, None]
    w1_ref[...] = p2[:, None]


def _sc_gather_body(x_hbm, src_hbm, xs_hbm, idx_v, rows_a, rows_b,
                    sem_a, sem_b):
    wid = lax.axis_index("s") * NC + lax.axis_index("c")
    rows_per_w = xs_hbm.shape[0] // NW
    nch = rows_per_w // GCH
    base = wid * rows_per_w
    pltpu.sync_copy(src_hbm.at[pl.ds(base, rows_per_w)], idx_v)
    bufs = (rows_a, rows_b)
    sems = (sem_a, sem_b)
    for b in range(2):
        pltpu.async_copy(
            x_hbm.at[idx_v.at[pl.ds(b * GCH, GCH)]], bufs[b], sems[b])

    def step(g, carry):
        for b in range(2):
            ch = g * 2 + b
            pltpu.make_async_copy(
                x_hbm.at[pl.ds(0, GCH)], bufs[b], sems[b]).wait()
            pltpu.sync_copy(bufs[b], xs_hbm.at[pl.ds(base + ch * GCH, GCH)])

            @pl.when(ch + 2 < nch)
            def _():
                pltpu.async_copy(
                    x_hbm.at[idx_v.at[pl.ds((ch + 2) * GCH, GCH)]],
                    bufs[b], sems[b])
        return carry

    lax.fori_loop(0, nch // 2, step, 0)


def _mm_body(be_ref, xs_ref, w1_ref, b1_ref, wg_ref, bg_ref, w2_ref, b2_ref,
             ws_ref, ys_ref):
    xb = xs_ref[...]
    h1 = lax.dot_general(
        xb, w1_ref[0], (((1,), (1,)), ((), ())),
        preferred_element_type=jnp.float32) + b1_ref[0]
    hg = lax.dot_general(
        xb, wg_ref[0], (((1,), (1,)), ((), ())),
        preferred_element_type=jnp.float32) + bg_ref[0]
    h = ((h1 * jax.nn.sigmoid(h1)) * hg).astype(jnp.bfloat16)
    eo = lax.dot_general(
        h, w2_ref[0], (((1,), (1,)), ((), ())),
        preferred_element_type=jnp.float32) + b2_ref[0]
    ys_ref[...] = eo * ws_ref[...]


def _sc_combine_body(ys_hbm, pos_hbm, out_hbm, pidx_v, in_a, in_b, out_v,
                     sem_a, sem_b):
    wid = lax.axis_index("s") * NC + lax.axis_index("c")
    tok_per_w = out_hbm.shape[0] // NW
    nch = tok_per_w // CCH
    tbase = wid * tok_per_w
    pltpu.sync_copy(pos_hbm.at[pl.ds(tbase * 2, tok_per_w * 2)], pidx_v)
    bufs = (in_a, in_b)
    sems = (sem_a, sem_b)
    for b in range(2):
        pltpu.async_copy(
            ys_hbm.at[pidx_v.at[pl.ds(b * 2 * CCH, 2 * CCH)]], bufs[b],
            sems[b])

    def step(g, carry):
        for b in range(2):
            ch = g * 2 + b
            pltpu.make_async_copy(
                ys_hbm.at[pl.ds(0, 2 * CCH)], bufs[b], sems[b]).wait()
            buf = bufs[b]

            def row_add(r, c2):
                for k in range(C // 16):
                    sl = pl.ds(k * 16, 16)
                    out_v[r, sl] = buf[2 * r, sl] + buf[2 * r + 1, sl]
                return c2

            lax.fori_loop(0, CCH, row_add, 0)
            pltpu.sync_copy(out_v, out_hbm.at[pl.ds(tbase + ch * CCH, CCH)])

            @pl.when(ch + 2 < nch)
            def _():
                pltpu.async_copy(
                    ys_hbm.at[pidx_v.at[pl.ds((ch + 2) * 2 * CCH, 2 * CCH)]],
                    bufs[b], sems[b])
        return carry

    lax.fori_loop(0, nch // 2, step, 0)


@jax.jit
def kernel(x, Wr, br, W1, b1, Wg, bg, W2, b2):
    B, T, _ = x.shape
    N = B * T
    P = N * TOPK
    NB = P // BLK + E
    NPAD = NB * BLK
    x2 = x.reshape(N, C)

    e0, e1, w0, w1 = pl.pallas_call(
        _router_body,
        grid=(N // BT_R,),
        in_specs=[
            pl.BlockSpec((BT_R, C), lambda i: (i, 0)),
            pl.BlockSpec((E, C), lambda i: (0, 0)),
            pl.BlockSpec((1, E), lambda i: (0, 0)),
        ],
        out_specs=[
            pl.BlockSpec((BT_R, 1), lambda i: (i, 0)),
            pl.BlockSpec((BT_R, 1), lambda i: (i, 0)),
            pl.BlockSpec((BT_R, 1), lambda i: (i, 0)),
            pl.BlockSpec((BT_R, 1), lambda i: (i, 0)),
        ],
        out_shape=[
            jax.ShapeDtypeStruct((N, 1), jnp.int32),
            jax.ShapeDtypeStruct((N, 1), jnp.int32),
            jax.ShapeDtypeStruct((N, 1), jnp.float32),
            jax.ShapeDtypeStruct((N, 1), jnp.float32),
        ],
    )(x2, Wr, br.reshape(1, E))

    e_flat = jnp.concatenate([e0, e1], axis=1).reshape(P)
    w_flat = jnp.concatenate([w0, w1], axis=1).reshape(P)
    oh = (e_flat[:, None] == jnp.arange(E, dtype=jnp.int32)[None, :])
    csum = jnp.cumsum(oh.astype(jnp.int32), axis=0)
    counts = csum[-1]
    rank = jnp.take_along_axis(csum, e_flat[:, None], axis=1)[:, 0] - 1
    pc = ((counts + BLK - 1) // BLK) * BLK
    ends = jnp.cumsum(pc)
    po = ends - pc
    pos = (po[e_flat] + rank).astype(jnp.int32)
    src = jnp.zeros((NPAD,), jnp.int32).at[pos].set(
        jnp.arange(P, dtype=jnp.int32) // TOPK)
    wsort = jnp.zeros((NPAD,), jnp.float32).at[pos].set(w_flat)
    bstarts = jnp.arange(NB, dtype=jnp.int32) * BLK
    be = jnp.minimum(
        jnp.searchsorted(ends, bstarts, side='right').astype(jnp.int32), E - 1)

    mesh = plsc.VectorSubcoreMesh(core_axis_name="c", subcore_axis_name="s", num_cores=NC, num_subcores=NS)
    x_pk = lax.bitcast_convert_type(
        x2.astype(jnp.bfloat16).reshape(N, C // 2, 2), jnp.float32)
    xs = pl.kernel(
        _sc_gather_body,
        out_type=jax.ShapeDtypeStruct((NPAD, C // 2), jnp.float32),
        mesh=mesh,
        scratch_types=[
            pltpu.VMEM((NPAD // NW,), jnp.int32),
            pltpu.VMEM((GCH, C // 2), jnp.float32),
            pltpu.VMEM((GCH, C // 2), jnp.float32),
            pltpu.SemaphoreType.DMA,
            pltpu.SemaphoreType.DMA,
        ],
    )(x_pk, src)
    xs_bf = lax.bitcast_convert_type(xs, jnp.bfloat16).reshape(NPAD, C)

    grid_spec = pltpu.PrefetchScalarGridSpec(
        num_scalar_prefetch=1,
        grid=(NB,),
        in_specs=[
            pl.BlockSpec((BLK, C), lambda i, be: (i, 0)),
            pl.BlockSpec((1, INNER, C), lambda i, be: (be[i], 0, 0)),
            pl.BlockSpec((1, 1, INNER), lambda i, be: (be[i], 0, 0)),
            pl.BlockSpec((1, INNER, C), lambda i, be: (be[i], 0, 0)),
            pl.BlockSpec((1, 1, INNER), lambda i, be: (be[i], 0, 0)),
            pl.BlockSpec((1, C, INNER), lambda i, be: (be[i], 0, 0)),
            pl.BlockSpec((1, 1, C), lambda i, be: (be[i], 0, 0)),
            pl.BlockSpec((BLK, 1), lambda i, be: (i, 0)),
        ],
        out_specs=pl.BlockSpec((BLK, C), lambda i, be: (i, 0)),
    )
    ys = pl.pallas_call(
        _mm_body,
        grid_spec=grid_spec,
        out_shape=jax.ShapeDtypeStruct((NPAD, C), jnp.float32),
        compiler_params=pltpu.CompilerParams(
            dimension_semantics=("arbitrary",)),
    )(be, xs_bf, W1.astype(jnp.bfloat16), b1.reshape(E, 1, INNER),
      Wg.astype(jnp.bfloat16), bg.reshape(E, 1, INNER),
      W2.astype(jnp.bfloat16), b2.reshape(E, 1, C), wsort.reshape(NPAD, 1))

    out = pl.kernel(
        _sc_combine_body,
        out_type=jax.ShapeDtypeStruct((N, C), jnp.float32),
        mesh=plsc.VectorSubcoreMesh(core_axis_name="c", subcore_axis_name="s", num_cores=NC, num_subcores=NS),
        scratch_types=[
            pltpu.VMEM((2 * N // NW,), jnp.int32),
            pltpu.VMEM((2 * CCH, C), jnp.float32),
            pltpu.VMEM((2 * CCH, C), jnp.float32),
            pltpu.VMEM((CCH, C), jnp.float32),
            pltpu.SemaphoreType.DMA,
            pltpu.SemaphoreType.DMA,
        ],
    )(ys, pos)

    return out.reshape(B, T, C)


# speedup vs baseline: 2.0303x; 2.0303x over previous
import functools

import jax
import jax.numpy as jnp
from jax import lax
from jax.experimental import pallas as pl
from jax.experimental.pallas import tpu as pltpu
from jax.experimental.pallas import tpu_sc as plsc

E = 8
TOPK = 2
C = 1024
INNER = 1024

BLK = 256
BT_R = 1024

NC, NS = 2, 16
NW = NC * NS
GCH = 48
CCH = 16


def _router_body(x_ref, wr_ref, br_ref, e0_ref, e1_ref, w0_ref, w1_ref):
    xb = x_ref[...]
    logits = lax.dot_general(
        xb, wr_ref[...], (((1,), (1,)), ((), ())),
        preferred_element_type=jnp.float32) + br_ref[...]
    p = jax.nn.softmax(logits, axis=-1)
    iota_e = lax.broadcasted_iota(jnp.int32, p.shape, 1)
    c1 = jnp.argmax(p, axis=-1)
    p1 = jnp.max(p, axis=-1)
    p_m = jnp.where(iota_e == c1[:, None], -jnp.inf, p)
    c2 = jnp.argmax(p_m, axis=-1)
    p2 = jnp.max(p_m, axis=-1)
    e0_ref[...] = c1[:, None].astype(jnp.int32)
    e1_ref[...] = c2[:, None].astype(jnp.int32)
    w0_ref[...] = p1[:, None]
    w1_ref[...] = p2[:, None]


def _sc_gather_body(x_hbm, src_hbm, xs_hbm, idx_v, rows_a, rows_b,
                    sem_a, sem_b):
    wid = lax.axis_index("s") * NC + lax.axis_index("c")
    rows_per_w = xs_hbm.shape[0] // NW
    nch = rows_per_w // GCH
    base = wid * rows_per_w
    pltpu.sync_copy(src_hbm.at[pl.ds(base, rows_per_w)], idx_v)
    bufs = (rows_a, rows_b)
    sems = (sem_a, sem_b)
    for b in range(2):
        pltpu.async_copy(
            x_hbm.at[idx_v.at[pl.ds(b * GCH, GCH)]], bufs[b], sems[b])

    def step(g, carry):
        for b in range(2):
            ch = g * 2 + b
            pltpu.make_async_copy(
                x_hbm.at[pl.ds(0, GCH)], bufs[b], sems[b]).wait()
            pltpu.sync_copy(bufs[b], xs_hbm.at[pl.ds(base + ch * GCH, GCH)])

            @pl.when(ch + 2 < nch)
            def _():
                pltpu.async_copy(
                    x_hbm.at[idx_v.at[pl.ds((ch + 2) * GCH, GCH)]],
                    bufs[b], sems[b])
        return carry

    lax.fori_loop(0, nch // 2, step, 0)


def _mm_body(be_ref, xs_ref, w1_ref, b1_ref, wg_ref, bg_ref, w2_ref, b2_ref,
             ws_ref, ys_ref):
    xb = xs_ref[...].astype(jnp.bfloat16)
    h1 = lax.dot_general(
        xb, w1_ref[0].astype(jnp.bfloat16), (((1,), (1,)), ((), ())),
        preferred_element_type=jnp.float32) + b1_ref[0]
    hg = lax.dot_general(
        xb, wg_ref[0].astype(jnp.bfloat16), (((1,), (1,)), ((), ())),
        preferred_element_type=jnp.float32) + bg_ref[0]
    h = ((h1 * jax.nn.sigmoid(h1)) * hg).astype(jnp.bfloat16)
    eo = lax.dot_general(
        h, w2_ref[0].astype(jnp.bfloat16), (((1,), (1,)), ((), ())),
        preferred_element_type=jnp.float32) + b2_ref[0]
    ys_ref[...] = eo * ws_ref[...]


def _sc_combine_body(ys_hbm, pos_hbm, out_hbm, pidx_v, in_a, in_b, out_v,
                     sem_a, sem_b):
    wid = lax.axis_index("s") * NC + lax.axis_index("c")
    tok_per_w = out_hbm.shape[0] // NW
    nch = tok_per_w // CCH
    tbase = wid * tok_per_w
    pltpu.sync_copy(pos_hbm.at[pl.ds(tbase * 2, tok_per_w * 2)], pidx_v)
    bufs = (in_a, in_b)
    sems = (sem_a, sem_b)
    for b in range(2):
        pltpu.async_copy(
            ys_hbm.at[pidx_v.at[pl.ds(b * 2 * CCH, 2 * CCH)]], bufs[b],
            sems[b])

    def step(g, carry):
        for b in range(2):
            ch = g * 2 + b
            pltpu.make_async_copy(
                ys_hbm.at[pl.ds(0, 2 * CCH)], bufs[b], sems[b]).wait()
            buf = bufs[b]

            def row_add(r, c2):
                for k in range(C // 16):
                    sl = pl.ds(k * 16, 16)
                    out_v[r, sl] = buf[2 * r, sl] + buf[2 * r + 1, sl]
                return c2

            lax.fori_loop(0, CCH, row_add, 0)
            pltpu.sync_copy(out_v, out_hbm.at[pl.ds(tbase + ch * CCH, CCH)])

            @pl.when(ch + 2 < nch)
            def _():
                pltpu.async_copy(
                    ys_hbm.at[pidx_v.at[pl.ds((ch + 2) * 2 * CCH, 2 * CCH)]],
                    bufs[b], sems[b])
        return carry

    lax.fori_loop(0, nch // 2, step, 0)


@jax.jit
def kernel(x, Wr, br, W1, b1, Wg, bg, W2, b2):
    B, T, _ = x.shape
    N = B * T
    P = N * TOPK
    NB = P // BLK + E
    NPAD = NB * BLK
    x2 = x.reshape(N, C)

    e0, e1, w0, w1 = pl.pallas_call(
        _router_body,
        grid=(N // BT_R,),
        in_specs=[
            pl.BlockSpec((BT_R, C), lambda i: (i, 0)),
            pl.BlockSpec((E, C), lambda i: (0, 0)),
            pl.BlockSpec((1, E), lambda i: (0, 0)),
        ],
        out_specs=[
            pl.BlockSpec((BT_R, 1), lambda i: (i, 0)),
            pl.BlockSpec((BT_R, 1), lambda i: (i, 0)),
            pl.BlockSpec((BT_R, 1), lambda i: (i, 0)),
            pl.BlockSpec((BT_R, 1), lambda i: (i, 0)),
        ],
        out_shape=[
            jax.ShapeDtypeStruct((N, 1), jnp.int32),
            jax.ShapeDtypeStruct((N, 1), jnp.int32),
            jax.ShapeDtypeStruct((N, 1), jnp.float32),
            jax.ShapeDtypeStruct((N, 1), jnp.float32),
        ],
    )(x2, Wr, br.reshape(1, E))

    e_flat = jnp.concatenate([e0, e1], axis=1).reshape(P)
    w_flat = jnp.concatenate([w0, w1], axis=1).reshape(P)
    oh = (e_flat[:, None] == jnp.arange(E, dtype=jnp.int32)[None, :])
    csum = jnp.cumsum(oh.astype(jnp.int32), axis=0)
    counts = csum[-1]
    rank = jnp.take_along_axis(csum, e_flat[:, None], axis=1)[:, 0] - 1
    pc = ((counts + BLK - 1) // BLK) * BLK
    ends = jnp.cumsum(pc)
    po = ends - pc
    pos = (po[e_flat] + rank).astype(jnp.int32)
    src = jnp.zeros((NPAD,), jnp.int32).at[pos].set(
        jnp.arange(P, dtype=jnp.int32) // TOPK)
    wsort = jnp.zeros((NPAD,), jnp.float32).at[pos].set(w_flat)
    bstarts = jnp.arange(NB, dtype=jnp.int32) * BLK
    be = jnp.minimum(
        jnp.searchsorted(ends, bstarts, side='right').astype(jnp.int32), E - 1)

    mesh = plsc.VectorSubcoreMesh(core_axis_name="c", subcore_axis_name="s", num_cores=NC, num_subcores=NS)
    xs = pl.kernel(
        _sc_gather_body,
        out_type=jax.ShapeDtypeStruct((NPAD, C), jnp.float32),
        mesh=mesh,
        scratch_types=[
            pltpu.VMEM((NPAD // NW,), jnp.int32),
            pltpu.VMEM((GCH, C), jnp.float32),
            pltpu.VMEM((GCH, C), jnp.float32),
            pltpu.SemaphoreType.DMA,
            pltpu.SemaphoreType.DMA,
        ],
    )(x2, src)

    grid_spec = pltpu.PrefetchScalarGridSpec(
        num_scalar_prefetch=1,
        grid=(NB,),
        in_specs=[
            pl.BlockSpec((BLK, C), lambda i, be: (i, 0)),
            pl.BlockSpec((1, INNER, C), lambda i, be: (be[i], 0, 0)),
            pl.BlockSpec((1, 1, INNER), lambda i, be: (be[i], 0, 0)),
            pl.BlockSpec((1, INNER, C), lambda i, be: (be[i], 0, 0)),
            pl.BlockSpec((1, 1, INNER), lambda i, be: (be[i], 0, 0)),
            pl.BlockSpec((1, C, INNER), lambda i, be: (be[i], 0, 0)),
            pl.BlockSpec((1, 1, C), lambda i, be: (be[i], 0, 0)),
            pl.BlockSpec((BLK, 1), lambda i, be: (i, 0)),
        ],
        out_specs=pl.BlockSpec((BLK, C), lambda i, be: (i, 0)),
    )
    ys = pl.pallas_call(
        _mm_body,
        grid_spec=grid_spec,
        out_shape=jax.ShapeDtypeStruct((NPAD, C), jnp.float32),
        compiler_params=pltpu.CompilerParams(
            dimension_semantics=("arbitrary",)),
    )(be, xs, W1, b1.reshape(E, 1, INNER), Wg, bg.reshape(E, 1, INNER),
      W2, b2.reshape(E, 1, C), wsort.reshape(NPAD, 1))

    out = pl.kernel(
        _sc_combine_body,
        out_type=jax.ShapeDtypeStruct((N, C), jnp.float32),
        mesh=plsc.VectorSubcoreMesh(core_axis_name="c", subcore_axis_name="s", num_cores=NC, num_subcores=NS),
        scratch_types=[
            pltpu.VMEM((2 * N // NW,), jnp.int32),
            pltpu.VMEM((2 * CCH, C), jnp.float32),
            pltpu.VMEM((2 * CCH, C), jnp.float32),
            pltpu.VMEM((CCH, C), jnp.float32),
            pltpu.SemaphoreType.DMA,
            pltpu.SemaphoreType.DMA,
        ],
    )(ys, pos)

    return out.reshape(B, T, C)
